# baseline (speedup 1.0000x reference)
import functools

import numpy as np
import jax
import jax.numpy as jnp
from jax import lax
from jax.experimental import pallas as pl
from jax.experimental.pallas import tpu as pltpu
from jax.experimental.pallas import tpu_sc as plsc

F32 = jnp.float32

BN = 400
BE = 4000
CHUNK = 100
GRP = 4
NC, NS = 2, 16


def _silu(v):
    return v * jax.nn.sigmoid(v)


def _outer(y, z):
    dz = z.shape[1]
    return jnp.concatenate([y[:, j:j + 1] * z for j in range(y.shape[1])],
                           axis=1)


def _embed_a1_body(x_ref, nat_ref, cat_ref, emw_ref, emaw_ref, emcw_ref,
                   wa1_ref, ycat_ref, s1_ref, h1_ref):
    z = jnp.dot(x_ref[...], emw_ref[...], preferred_element_type=F32,
                 precision=jax.lax.Precision.HIGHEST)
    na = jnp.dot(nat_ref[...], emaw_ref[...], preferred_element_type=F32,
                 precision=jax.lax.Precision.HIGHEST)
    ca = jnp.dot(cat_ref[...], emcw_ref[...], preferred_element_type=F32,
                 precision=jax.lax.Precision.HIGHEST)
    y = jnp.concatenate([na, ca], axis=1)
    ycat_ref[...] = y
    p = _outer(y, z)
    sh = jnp.dot(p, wa1_ref[...], preferred_element_type=F32,
                 precision=jax.lax.Precision.HIGHEST)
    s1_ref[...] = sh[:, :64]
    pad = jnp.zeros((sh.shape[0], 96), F32)
    h1_ref[0] = jnp.concatenate([sh[:, 64:96], pad], axis=1)
    h1_ref[1] = jnp.concatenate([sh[:, 96:128], pad], axis=1)


def _radial_body(ele_ref, w0_ref, w1blk_ref, w1_ref, w2_ref, w3_ref):
    h = jnp.dot(ele_ref[...], w0_ref[...], preferred_element_type=F32,
                 precision=jax.lax.Precision.HIGHEST)
    h = _silu(h)
    wall = jnp.dot(h, w1blk_ref[...], preferred_element_type=F32,
                 precision=jax.lax.Precision.HIGHEST)
    nq = wall.shape[0] // (4 * CHUNK)
    for l, wref in enumerate((w1_ref, w2_ref, w3_ref)):
        for c in range(2):
            col = 64 * l + 32 * c
            pieces = [
                jnp.concatenate(
                    [wall[q * 4 * CHUNK + t * CHUNK:
                          q * 4 * CHUNK + (t + 1) * CHUNK, col:col + 32]
                     for q in range(nq)], axis=0)
                for t in range(4)
            ]
            wref[c] = jnp.concatenate(pieces, axis=1)


def _conv_node_body(agg_ref, s_ref, ycat_ref, wbl3_ref, wanext_ref,
                    snext_ref, hnext_ref, *, s_width):
    y = ycat_ref[...]
    agg = jnp.concatenate([agg_ref[0], agg_ref[1]], axis=1)
    pa = _outer(y, agg)
    h2 = jnp.dot(pa, wbl3_ref[...], preferred_element_type=F32,
                 precision=jax.lax.Precision.HIGHEST)
    zn = _silu(s_ref[...] + h2)
    pz = _outer(y, zn)
    sh = jnp.dot(pz, wanext_ref[...], preferred_element_type=F32,
                 precision=jax.lax.Precision.HIGHEST)
    snext_ref[...] = sh[:, :s_width]
    pad = jnp.zeros((sh.shape[0], 96), F32)
    hnext_ref[0] = jnp.concatenate([sh[:, s_width:s_width + 32], pad], axis=1)
    hnext_ref[1] = jnp.concatenate([sh[:, s_width + 32:s_width + 64], pad],
                                   axis=1)


def _final_body(agg_ref, s3_ref, ycat_ref, wbl3_ref, out_ref, *, inv_sqrt_n):
    i = pl.program_id(0)
    y = ycat_ref[...]
    agg = jnp.concatenate([agg_ref[0], agg_ref[1]], axis=1)
    pa = _outer(y, agg)
    h2 = jnp.dot(pa, wbl3_ref[...], preferred_element_type=F32,
                 precision=jax.lax.Precision.HIGHEST)
    z3 = s3_ref[...] + h2
    part = jnp.sum(z3) * inv_sqrt_n

    @pl.when(i == 0)
    def _():
        out_ref[...] = jnp.zeros_like(out_ref)

    out_ref[...] += part[None, None]


def _edge_sc_body(n_nodes, n_edges,
                  h_hbm, w_hbm, srcm_hbm, dstm_hbm, agg_hbm,
                  rows_v, wv, prod_v, idxs_v, idxd_v, gsem, wsem, agg_sh):
    c = lax.axis_index("c")
    s = lax.axis_index("s")
    rows_per_tile = n_edges // CHUNK // NS
    n_groups = rows_per_tile // GRP
    tile_row0 = s * rows_per_tile
    stripe = n_nodes // NS

    def _zero_body(r, _):
        prod_v[r, 0:16] = jnp.zeros((16,), F32)
        prod_v[r, 16:32] = jnp.zeros((16,), F32)
        return _
    lax.fori_loop(0, CHUNK, _zero_body, 0)
    nz = stripe // CHUNK
    rem = stripe - nz * CHUNK
    for k in range(nz):
        pltpu.sync_copy(prod_v.at[pl.ds(0, CHUNK)],
                        agg_sh.at[pl.ds(s * stripe + k * CHUNK, CHUNK)])
    if rem:
        pltpu.sync_copy(prod_v.at[pl.ds(0, rem)],
                        agg_sh.at[pl.ds(s * stripe + nz * CHUNK, rem)])
    plsc.subcore_barrier()

    def _group(g, _):
        row = tile_row0 + g * GRP
        pltpu.sync_copy(srcm_hbm.at[c, pl.ds(row, GRP)], idxs_v)
        pltpu.sync_copy(dstm_hbm.at[pl.ds(row, GRP)], idxd_v)
        wrow = c * (n_edges // 4) + (row * CHUNK) // 4
        wdesc = pltpu.async_copy(w_hbm.at[pl.ds(wrow, CHUNK)], wv, wsem)
        wdesc.wait()
        for ci in range(GRP):
            pltpu.async_copy(h_hbm.at[idxs_v.at[ci]], rows_v, gsem).wait()

            def _mul(r, _):
                prod_v[r, 0:16] = (rows_v[r, 0:16]
                                   * wv[r, ci * 32:ci * 32 + 16])
                prod_v[r, 16:32] = (rows_v[r, 16:32]
                                    * wv[r, ci * 32 + 16:ci * 32 + 32])
                return _
            lax.fori_loop(0, CHUNK, _mul, 0)
            pltpu.sync_copy(prod_v, agg_sh.at[idxd_v.at[ci]], add=True)
        return _

    lax.fori_loop(0, n_groups, _group, 0)
    plsc.subcore_barrier()

    pltpu.sync_copy(agg_sh.at[pl.ds(s * stripe, stripe)],
                    agg_hbm.at[c, pl.ds(s * stripe, stripe)])


def _edge_aggregate(ht, wpk, srcm2, dstm, n_nodes, n_edges):
    mesh = plsc.VectorSubcoreMesh(core_axis_name="c", subcore_axis_name="s",
                                  num_cores=NC, num_subcores=NS)
    body = functools.partial(_edge_sc_body, n_nodes, n_edges)
    return pl.kernel(
        body,
        out_type=jax.ShapeDtypeStruct((2, n_nodes, 32), F32),
        mesh=mesh,
        compiler_params=pltpu.CompilerParams(use_tc_tiling_on_sc=False),
        scratch_types=[
            pltpu.VMEM((CHUNK, 128), F32),
            pltpu.VMEM((CHUNK, 128), F32),
            pltpu.VMEM((CHUNK, 32), F32),
            pltpu.VMEM((GRP, CHUNK), jnp.int32),
            pltpu.VMEM((GRP, CHUNK), jnp.int32),
            pltpu.SemaphoreType.DMA,
            pltpu.SemaphoreType.DMA,
            pltpu.VMEM_SHARED((n_nodes, 32), F32),
        ],
    )(ht, wpk.reshape(2 * (n_edges // 4), 128), srcm2, dstm)


def _prep_layer(p, c_s, c_x, d_out):
    d_in = p['l1_a'].shape[0]
    norm = 1.0 / np.sqrt(d_in * 16.0)
    wsc = jnp.concatenate([p['sc_a'], p['sc_c']], axis=1) * (c_s * norm)
    wl1 = jnp.concatenate([p['l1_a'], p['l1_c']], axis=1) * norm
    wa = jnp.concatenate([wsc, wl1], axis=2)
    wa = jnp.transpose(wa, (1, 0, 2)).reshape(32 * d_in, d_out + 64)
    wb = jnp.concatenate([p['l2_a'], p['l2_c']], axis=1) * norm
    wb = jnp.transpose(wb, (1, 0, 2)).reshape(32 * d_in, d_in)
    l3s = jnp.sum(p['l3'], axis=1) * (c_x * norm)
    wbl3 = jnp.dot(wb, l3s)
    return wa, wbl3


def kernel(x, node_attr, crystal_attr, sym_mask, edge_attr,
           edge_length_embedded, params, edge_src, edge_dst):
    n, in_dim = x.shape
    e = edge_src.shape[0]
    c_s = np.sin(np.pi / 8.0)
    c_x = np.cos(np.pi / 8.0)
    layers = params['layers']

    wa1, wbl31 = _prep_layer(layers[0], c_s, c_x, 64)
    wa2, wbl32 = _prep_layer(layers[1], c_s, c_x, 64)
    wa3, wbl33 = _prep_layer(layers[2], c_s, c_x, 1)

    w0all = jnp.concatenate([layers[0]['fc0'], layers[1]['fc0'],
                             layers[2]['fc0']], axis=1) / np.sqrt(10.0)
    w1blk = jnp.zeros((300, 192), F32)
    for l in range(3):
        w1blk = w1blk.at[100 * l:100 * (l + 1), 64 * l:64 * (l + 1)].set(
            layers[l]['fc1'] / (10.0 * 4.0))

    grid_n = n // BN
    grid_e = e // BE

    full = lambda shape: pl.BlockSpec(shape, lambda i: tuple(0 for _ in shape))

    ycat, s1, h1 = pl.pallas_call(
        _embed_a1_body,
        grid=(grid_n,),
        in_specs=[
            pl.BlockSpec((BN, in_dim), lambda i: (i, 0)),
            pl.BlockSpec((BN, 16), lambda i: (i, 0)),
            pl.BlockSpec((BN, 16), lambda i: (i, 0)),
            full((in_dim, 64)), full((16, 16)), full((16, 16)),
            full((2048, 128)),
        ],
        out_specs=[
            pl.BlockSpec((BN, 32), lambda i: (i, 0)),
            pl.BlockSpec((BN, 64), lambda i: (i, 0)),
            pl.BlockSpec((2, BN, 128), lambda i: (0, i, 0)),
        ],
        out_shape=[
            jax.ShapeDtypeStruct((n, 32), F32),
            jax.ShapeDtypeStruct((n, 64), F32),
            jax.ShapeDtypeStruct((2, n, 128), F32),
        ],
    )(x, node_attr, crystal_attr, params['em_w'], params['ema_w'],
      params['emc_w'], wa1)

    w1, w2, w3 = pl.pallas_call(
        _radial_body,
        grid=(grid_e,),
        in_specs=[
            pl.BlockSpec((BE, 10), lambda i: (i, 0)),
            full((10, 300)), full((300, 192)),
        ],
        out_specs=[pl.BlockSpec((2, BE // 4, 128), lambda i: (0, i, 0))] * 3,
        out_shape=[jax.ShapeDtypeStruct((2, e // 4, 128), F32)] * 3,
    )(edge_length_embedded, w0all, w1blk)

    src32 = edge_src.astype(jnp.int32)
    srcm2 = jnp.stack([src32, src32 + n]).reshape(2, e // CHUNK, CHUNK)
    dstm = jnp.reshape(edge_dst.astype(jnp.int32), (e // CHUNK, CHUNK))

    def conv_node(agg, s_prev, wbl3, wanext, s_width):
        return pl.pallas_call(
            functools.partial(_conv_node_body, s_width=s_width),
            grid=(grid_n,),
            in_specs=[
                pl.BlockSpec((2, BN, 32), lambda i: (0, i, 0)),
                pl.BlockSpec((BN, 64), lambda i: (i, 0)),
                pl.BlockSpec((BN, 32), lambda i: (i, 0)),
                full((2048, 64)), full((2048, s_width + 64)),
            ],
            out_specs=[
                pl.BlockSpec((BN, s_width), lambda i: (i, 0)),
                pl.BlockSpec((2, BN, 128), lambda i: (0, i, 0)),
            ],
            out_shape=[
                jax.ShapeDtypeStruct((n, s_width), F32),
                jax.ShapeDtypeStruct((2, n, 128), F32),
            ],
        )(agg, s_prev, ycat, wbl3, wanext)

    agg = _edge_aggregate(h1.reshape(2 * n, 128), w1,
                          srcm2, dstm, n, e)
    s2, h2 = conv_node(agg, s1, wbl31, wa2, 64)

    agg = _edge_aggregate(h2.reshape(2 * n, 128), w2,
                          srcm2, dstm, n, e)
    s3, h3 = conv_node(agg, s2, wbl32, wa3, 1)

    agg = _edge_aggregate(h3.reshape(2 * n, 128), w3,
                          srcm2, dstm, n, e)
    out = pl.pallas_call(
        functools.partial(_final_body, inv_sqrt_n=1.0 / np.sqrt(float(n))),
        grid=(grid_n,),
        in_specs=[
            pl.BlockSpec((2, BN, 32), lambda i: (0, i, 0)),
            pl.BlockSpec((BN, 1), lambda i: (i, 0)),
            pl.BlockSpec((BN, 32), lambda i: (i, 0)),
            full((2048, 1)),
        ],
        out_specs=pl.BlockSpec((1, 1), lambda i: (0, 0)),
        out_shape=jax.ShapeDtypeStruct((1, 1), F32),
    )(agg, s3, ycat, wbl33)
    return out


# speedup vs baseline: 1.2556x; 1.2556x over previous
import functools

import numpy as np
import jax
import jax.numpy as jnp
from jax import lax
from jax.experimental import pallas as pl
from jax.experimental.pallas import tpu as pltpu
from jax.experimental.pallas import tpu_sc as plsc

F32 = jnp.float32

BN = 400
BE = 4000
CHUNK = 100
GRP = 4
NC, NS = 2, 16


def _silu(v):
    return v * jax.nn.sigmoid(v)


def _outer(y, z):
    dz = z.shape[1]
    return jnp.concatenate([y[:, j:j + 1] * z for j in range(y.shape[1])],
                           axis=1)


def _embed_a1_body(x_ref, nat_ref, cat_ref, emw_ref, emaw_ref, emcw_ref,
                   wa1_ref, ycat_ref, s1_ref, h1_ref):
    z = jnp.dot(x_ref[...], emw_ref[...], preferred_element_type=F32,
                 precision=jax.lax.Precision.HIGHEST)
    na = jnp.dot(nat_ref[...], emaw_ref[...], preferred_element_type=F32,
                 precision=jax.lax.Precision.HIGHEST)
    ca = jnp.dot(cat_ref[...], emcw_ref[...], preferred_element_type=F32,
                 precision=jax.lax.Precision.HIGHEST)
    y = jnp.concatenate([na, ca], axis=1)
    ycat_ref[...] = y
    p = _outer(y, z)
    sh = jnp.dot(p, wa1_ref[...], preferred_element_type=F32,
                 precision=jax.lax.Precision.HIGHEST)
    s1_ref[...] = sh[:, :64]
    h1_ref[0] = sh[:, 64:96]
    h1_ref[1] = sh[:, 96:128]


def _radial_body(ele_ref, w0_ref, w1blk_ref, w1_ref, w2_ref, w3_ref):
    h = jnp.dot(ele_ref[...], w0_ref[...], preferred_element_type=F32,
                 precision=jax.lax.Precision.HIGHEST)
    h = _silu(h)
    wall = jnp.dot(h, w1blk_ref[...], preferred_element_type=F32,
                 precision=jax.lax.Precision.HIGHEST)
    nq = wall.shape[0] // (4 * CHUNK)
    for l, wref in enumerate((w1_ref, w2_ref, w3_ref)):
        for c in range(2):
            col = 64 * l + 32 * c
            pieces = [
                jnp.concatenate(
                    [wall[q * 4 * CHUNK + t * CHUNK:
                          q * 4 * CHUNK + (t + 1) * CHUNK, col:col + 32]
                     for q in range(nq)], axis=0)
                for t in range(4)
            ]
            wref[c] = jnp.concatenate(pieces, axis=1)


def _conv_node_body(agg_ref, s_ref, ycat_ref, wbl3_ref, wanext_ref,
                    snext_ref, hnext_ref, *, s_width):
    y = ycat_ref[...]
    agg = jnp.concatenate([agg_ref[0], agg_ref[1]], axis=1)
    pa = _outer(y, agg)
    h2 = jnp.dot(pa, wbl3_ref[...], preferred_element_type=F32,
                 precision=jax.lax.Precision.HIGHEST)
    zn = _silu(s_ref[...] + h2)
    pz = _outer(y, zn)
    sh = jnp.dot(pz, wanext_ref[...], preferred_element_type=F32,
                 precision=jax.lax.Precision.HIGHEST)
    snext_ref[...] = sh[:, :s_width]
    hnext_ref[0] = sh[:, s_width:s_width + 32]
    hnext_ref[1] = sh[:, s_width + 32:s_width + 64]


def _final_body(agg_ref, s3_ref, ycat_ref, wbl3_ref, out_ref, *, inv_sqrt_n):
    i = pl.program_id(0)
    y = ycat_ref[...]
    agg = jnp.concatenate([agg_ref[0], agg_ref[1]], axis=1)
    pa = _outer(y, agg)
    h2 = jnp.dot(pa, wbl3_ref[...], preferred_element_type=F32,
                 precision=jax.lax.Precision.HIGHEST)
    z3 = s3_ref[...] + h2
    part = jnp.sum(z3) * inv_sqrt_n

    @pl.when(i == 0)
    def _():
        out_ref[...] = jnp.zeros_like(out_ref)

    out_ref[...] += part[None, None]


def _edge_sc_body(n_nodes, n_edges,
                  h_hbm, w_hbm, srcm_hbm, dstm_hbm, agg_hbm,
                  rows_v, wv, prod_v, idxs_v, idxd_v,
                  gsem, wsem, isem, ssem, agg_sh):
    c = lax.axis_index("c")
    s = lax.axis_index("s")
    rows_per_tile = n_edges // CHUNK // NS
    n_groups = rows_per_tile // GRP
    tile_row0 = s * rows_per_tile
    w_row0 = c * (n_edges // 4) + tile_row0 * (CHUNK // 4)
    gw = GRP * (CHUNK // 4)
    stripe = n_nodes // NS

    def _zero_body(r, _):
        prod_v[0, r, 0:16] = jnp.zeros((16,), F32)
        prod_v[0, r, 16:32] = jnp.zeros((16,), F32)
        return _
    lax.fori_loop(0, CHUNK, _zero_body, 0)
    nz = stripe // CHUNK
    rem = stripe - nz * CHUNK
    for k in range(nz):
        pltpu.sync_copy(prod_v.at[0, pl.ds(0, CHUNK)],
                        agg_sh.at[pl.ds(s * stripe + k * CHUNK, CHUNK)])
    if rem:
        pltpu.sync_copy(prod_v.at[0, pl.ds(0, rem)],
                        agg_sh.at[pl.ds(s * stripe + nz * CHUNK, rem)])
    plsc.subcore_barrier()

    def _idx_load(g, buf, sync):
        row = tile_row0 + g * GRP
        if sync:
            pltpu.sync_copy(srcm_hbm.at[c, pl.ds(row, GRP)], idxs_v.at[buf])
            pltpu.sync_copy(dstm_hbm.at[pl.ds(row, GRP)], idxd_v.at[buf])
        else:
            pltpu.async_copy(srcm_hbm.at[c, pl.ds(row, GRP)],
                             idxs_v.at[buf], isem)
            pltpu.async_copy(dstm_hbm.at[pl.ds(row, GRP)],
                             idxd_v.at[buf], isem)

    def _fire_gather(idxrow, rbuf):
        pltpu.async_copy(h_hbm.at[idxrow], rows_v.at[rbuf], gsem)

    def _drain(sem, dst):
        pltpu.make_async_copy(h_hbm.at[pl.ds(0, CHUNK)], dst, sem).wait()

    _idx_load(0, 0, True)
    @pl.when(n_groups > 1)
    def _():
        _idx_load(1, 1, False)
    pltpu.sync_copy(w_hbm.at[pl.ds(w_row0, gw)], wv)
    _fire_gather(idxs_v.at[0, 0], 0)

    def _group(g, carry):
        pg = lax.rem(g, 2)
        for ci in range(GRP):
            rb = ci & 1
            if ci < GRP - 1:
                _fire_gather(idxs_v.at[pg, ci + 1], 1 - rb)
            else:
                @pl.when(g < n_groups - 1)
                def _():
                    pltpu.make_async_copy(srcm_hbm.at[c, pl.ds(0, GRP)],
                                          idxs_v.at[0], isem).wait()
                    pltpu.make_async_copy(dstm_hbm.at[pl.ds(0, GRP)],
                                          idxd_v.at[0], isem).wait()
                    _fire_gather(idxs_v.at[1 - pg, 0], 1 - rb)
            _drain(gsem, rows_v.at[rb])
            if ci < 2:
                @pl.when(g > 0)
                def _():
                    pltpu.make_async_copy(
                        prod_v.at[rb], agg_sh.at[pl.ds(0, CHUNK)],
                        ssem).wait()
            else:
                pltpu.make_async_copy(prod_v.at[rb],
                                      agg_sh.at[pl.ds(0, CHUNK)], ssem).wait()

            def _mul(r, _):
                prod_v[rb, r, 0:16] = (rows_v[rb, r, 0:16]
                                       * wv[r, ci * 32:ci * 32 + 16])
                prod_v[rb, r, 16:32] = (rows_v[rb, r, 16:32]
                                        * wv[r, ci * 32 + 16:ci * 32 + 32])
                return _
            lax.fori_loop(0, CHUNK, _mul, 0)
            pltpu.async_copy(prod_v.at[rb], agg_sh.at[idxd_v.at[pg, ci]],
                             ssem, add=True)
        @pl.when(g < n_groups - 2)
        def _():
            _idx_load(g + 2, pg, False)
        @pl.when(g < n_groups - 1)
        def _():
            pltpu.sync_copy(w_hbm.at[pl.ds(w_row0 + (g + 1) * gw, gw)], wv)
        return carry

    lax.fori_loop(0, n_groups, _group, 0)
    for rb in range(2):
        pltpu.make_async_copy(prod_v.at[rb], agg_sh.at[pl.ds(0, CHUNK)],
                              ssem).wait()
    plsc.subcore_barrier()

    pltpu.sync_copy(agg_sh.at[pl.ds(s * stripe, stripe)],
                    agg_hbm.at[c, pl.ds(s * stripe, stripe)])


def _edge_aggregate(ht, wpk, srcm2, dstm, n_nodes, n_edges):
    mesh = plsc.VectorSubcoreMesh(core_axis_name="c", subcore_axis_name="s",
                                  num_cores=NC, num_subcores=NS)
    body = functools.partial(_edge_sc_body, n_nodes, n_edges)
    return pl.kernel(
        body,
        out_type=jax.ShapeDtypeStruct((2, n_nodes, 32), F32),
        mesh=mesh,
        compiler_params=pltpu.CompilerParams(use_tc_tiling_on_sc=False),
        scratch_types=[
            pltpu.VMEM((2, CHUNK, 32), F32),
            pltpu.VMEM((GRP * CHUNK // 4, 128), F32),
            pltpu.VMEM((2, CHUNK, 32), F32),
            pltpu.VMEM((2, GRP, CHUNK), jnp.int32),
            pltpu.VMEM((2, GRP, CHUNK), jnp.int32),
            pltpu.SemaphoreType.DMA,
            pltpu.SemaphoreType.DMA,
            pltpu.SemaphoreType.DMA,
            pltpu.SemaphoreType.DMA,
            pltpu.VMEM_SHARED((n_nodes, 32), F32),
        ],
    )(ht, wpk.reshape(2 * (n_edges // 4), 128), srcm2, dstm)


def _prep_layer(p, c_s, c_x, d_out):
    d_in = p['l1_a'].shape[0]
    norm = 1.0 / np.sqrt(d_in * 16.0)
    wsc = jnp.concatenate([p['sc_a'], p['sc_c']], axis=1) * (c_s * norm)
    wl1 = jnp.concatenate([p['l1_a'], p['l1_c']], axis=1) * norm
    wa = jnp.concatenate([wsc, wl1], axis=2)
    wa = jnp.transpose(wa, (1, 0, 2)).reshape(32 * d_in, d_out + 64)
    wb = jnp.concatenate([p['l2_a'], p['l2_c']], axis=1) * norm
    wb = jnp.transpose(wb, (1, 0, 2)).reshape(32 * d_in, d_in)
    l3s = jnp.sum(p['l3'], axis=1) * (c_x * norm)
    wbl3 = jnp.dot(wb, l3s)
    return wa, wbl3


def kernel(x, node_attr, crystal_attr, sym_mask, edge_attr,
           edge_length_embedded, params, edge_src, edge_dst):
    n, in_dim = x.shape
    e = edge_src.shape[0]
    c_s = np.sin(np.pi / 8.0)
    c_x = np.cos(np.pi / 8.0)
    layers = params['layers']

    wa1, wbl31 = _prep_layer(layers[0], c_s, c_x, 64)
    wa2, wbl32 = _prep_layer(layers[1], c_s, c_x, 64)
    wa3, wbl33 = _prep_layer(layers[2], c_s, c_x, 1)

    w0all = jnp.concatenate([layers[0]['fc0'], layers[1]['fc0'],
                             layers[2]['fc0']], axis=1) / np.sqrt(10.0)
    w1blk = jnp.zeros((300, 192), F32)
    for l in range(3):
        w1blk = w1blk.at[100 * l:100 * (l + 1), 64 * l:64 * (l + 1)].set(
            layers[l]['fc1'] / (10.0 * 4.0))

    grid_n = n // BN
    grid_e = e // BE

    full = lambda shape: pl.BlockSpec(shape, lambda i: tuple(0 for _ in shape))

    ycat, s1, h1 = pl.pallas_call(
        _embed_a1_body,
        grid=(grid_n,),
        in_specs=[
            pl.BlockSpec((BN, in_dim), lambda i: (i, 0)),
            pl.BlockSpec((BN, 16), lambda i: (i, 0)),
            pl.BlockSpec((BN, 16), lambda i: (i, 0)),
            full((in_dim, 64)), full((16, 16)), full((16, 16)),
            full((2048, 128)),
        ],
        out_specs=[
            pl.BlockSpec((BN, 32), lambda i: (i, 0)),
            pl.BlockSpec((BN, 64), lambda i: (i, 0)),
            pl.BlockSpec((2, BN, 32), lambda i: (0, i, 0)),
        ],
        out_shape=[
            jax.ShapeDtypeStruct((n, 32), F32),
            jax.ShapeDtypeStruct((n, 64), F32),
            jax.ShapeDtypeStruct((2, n, 32), F32),
        ],
    )(x, node_attr, crystal_attr, params['em_w'], params['ema_w'],
      params['emc_w'], wa1)

    w1, w2, w3 = pl.pallas_call(
        _radial_body,
        grid=(grid_e,),
        in_specs=[
            pl.BlockSpec((BE, 10), lambda i: (i, 0)),
            full((10, 300)), full((300, 192)),
        ],
        out_specs=[pl.BlockSpec((2, BE // 4, 128), lambda i: (0, i, 0))] * 3,
        out_shape=[jax.ShapeDtypeStruct((2, e // 4, 128), F32)] * 3,
    )(edge_length_embedded, w0all, w1blk)

    src32 = edge_src.astype(jnp.int32)
    srcm2 = jnp.stack([src32, src32 + n]).reshape(2, e // CHUNK, CHUNK)
    dstm = jnp.reshape(edge_dst.astype(jnp.int32), (e // CHUNK, CHUNK))

    def conv_node(agg, s_prev, wbl3, wanext, s_width):
        return pl.pallas_call(
            functools.partial(_conv_node_body, s_width=s_width),
            grid=(grid_n,),
            in_specs=[
                pl.BlockSpec((2, BN, 32), lambda i: (0, i, 0)),
                pl.BlockSpec((BN, 64), lambda i: (i, 0)),
                pl.BlockSpec((BN, 32), lambda i: (i, 0)),
                full((2048, 64)), full((2048, s_width + 64)),
            ],
            out_specs=[
                pl.BlockSpec((BN, s_width), lambda i: (i, 0)),
                pl.BlockSpec((2, BN, 32), lambda i: (0, i, 0)),
            ],
            out_shape=[
                jax.ShapeDtypeStruct((n, s_width), F32),
                jax.ShapeDtypeStruct((2, n, 32), F32),
            ],
        )(agg, s_prev, ycat, wbl3, wanext)

    agg = _edge_aggregate(h1.reshape(2 * n, 32), w1,
                          srcm2, dstm, n, e)
    s2, h2 = conv_node(agg, s1, wbl31, wa2, 64)

    agg = _edge_aggregate(h2.reshape(2 * n, 32), w2,
                          srcm2, dstm, n, e)
    s3, h3 = conv_node(agg, s2, wbl32, wa3, 1)

    agg = _edge_aggregate(h3.reshape(2 * n, 32), w3,
                          srcm2, dstm, n, e)
    out = pl.pallas_call(
        functools.partial(_final_body, inv_sqrt_n=1.0 / np.sqrt(float(n))),
        grid=(grid_n,),
        in_specs=[
            pl.BlockSpec((2, BN, 32), lambda i: (0, i, 0)),
            pl.BlockSpec((BN, 1), lambda i: (i, 0)),
            pl.BlockSpec((BN, 32), lambda i: (i, 0)),
            full((2048, 1)),
        ],
        out_specs=pl.BlockSpec((1, 1), lambda i: (0, 0)),
        out_shape=jax.ShapeDtypeStruct((1, 1), F32),
    )(agg, s3, ycat, wbl33)
    return out


# speedup vs baseline: 1.6004x; 1.2746x over previous
import functools

import numpy as np
import jax
import jax.numpy as jnp
from jax import lax
from jax.experimental import pallas as pl
from jax.experimental.pallas import tpu as pltpu
from jax.experimental.pallas import tpu_sc as plsc

F32 = jnp.float32

BN = 400
BE = 4000
CHUNK = 100
GRP = 4
NC, NS = 2, 16


BF16 = jnp.bfloat16


def _dot(a, b):
    ah = a.astype(BF16)
    al = (a - ah.astype(F32)).astype(BF16)
    bh = b.astype(BF16)
    bl = (b - bh.astype(F32)).astype(BF16)
    d = lambda u, v: lax.dot_general(u, v, (((1,), (0,)), ((), ())),
                                     preferred_element_type=F32)
    return d(ah, bh) + d(ah, bl) + d(al, bh)


def _silu(v):
    return v * jax.nn.sigmoid(v)


def _outer(y, z):
    dz = z.shape[1]
    return jnp.concatenate([y[:, j:j + 1] * z for j in range(y.shape[1])],
                           axis=1)


def _embed_a1_body(x_ref, nat_ref, cat_ref, emw_ref, emaw_ref, emcw_ref,
                   wa1_ref, ycat_ref, s1_ref, h1_ref):
    z = _dot(x_ref[...], emw_ref[...])
    na = _dot(nat_ref[...], emaw_ref[...])
    ca = _dot(cat_ref[...], emcw_ref[...])
    y = jnp.concatenate([na, ca], axis=1)
    ycat_ref[...] = y
    p = _outer(y, z)
    sh = _dot(p, wa1_ref[...])
    s1_ref[...] = sh[:, :64]
    h1_ref[0] = sh[:, 64:96]
    h1_ref[1] = sh[:, 96:128]


def _radial_body(ele_ref, w0_ref, w1blk_ref, w1_ref, w2_ref, w3_ref):
    h = _dot(ele_ref[...], w0_ref[...])
    h = _silu(h)
    wall = _dot(h, w1blk_ref[...])
    nq = wall.shape[0] // (4 * CHUNK)
    for l, wref in enumerate((w1_ref, w2_ref, w3_ref)):
        for c in range(2):
            col = 64 * l + 32 * c
            pieces = [
                jnp.concatenate(
                    [wall[q * 4 * CHUNK + t * CHUNK:
                          q * 4 * CHUNK + (t + 1) * CHUNK, col:col + 32]
                     for q in range(nq)], axis=0)
                for t in range(4)
            ]
            wref[c] = jnp.concatenate(pieces, axis=1)


def _conv_node_body(agg_ref, s_ref, ycat_ref, wbl3_ref, wanext_ref,
                    snext_ref, hnext_ref, *, s_width):
    y = ycat_ref[...]
    agg = jnp.concatenate([agg_ref[0], agg_ref[1]], axis=1)
    pa = _outer(y, agg)
    h2 = _dot(pa, wbl3_ref[...])
    zn = _silu(s_ref[...] + h2)
    pz = _outer(y, zn)
    sh = _dot(pz, wanext_ref[...])
    snext_ref[...] = sh[:, :s_width]
    hnext_ref[0] = sh[:, s_width:s_width + 32]
    hnext_ref[1] = sh[:, s_width + 32:s_width + 64]


def _final_body(agg_ref, s3_ref, ycat_ref, wbl3_ref, out_ref, *, inv_sqrt_n):
    i = pl.program_id(0)
    y = ycat_ref[...]
    agg = jnp.concatenate([agg_ref[0], agg_ref[1]], axis=1)
    pa = _outer(y, agg)
    h2 = _dot(pa, wbl3_ref[...])
    z3 = s3_ref[...] + h2
    part = jnp.sum(z3) * inv_sqrt_n

    @pl.when(i == 0)
    def _():
        out_ref[...] = jnp.zeros_like(out_ref)

    out_ref[...] += part[None, None]


def _edge_sc_body(n_nodes, n_edges,
                  h_hbm, w_hbm, srcm_hbm, dstm_hbm, agg_hbm,
                  rows_v, wv, prod_v, idxs_v, idxd_v,
                  gsem, wsem, isem, ssem, agg_sh):
    c = lax.axis_index("c")
    s = lax.axis_index("s")
    rows_per_tile = n_edges // CHUNK // NS
    n_groups = rows_per_tile // GRP
    tile_row0 = s * rows_per_tile
    w_row0 = c * (n_edges // 4) + tile_row0 * (CHUNK // 4)
    gw = GRP * (CHUNK // 4)
    stripe = n_nodes // NS

    def _zero_body(r, _):
        prod_v[0, r, 0:16] = jnp.zeros((16,), F32)
        prod_v[0, r, 16:32] = jnp.zeros((16,), F32)
        return _
    lax.fori_loop(0, CHUNK, _zero_body, 0)
    nz = stripe // CHUNK
    rem = stripe - nz * CHUNK
    for k in range(nz):
        pltpu.sync_copy(prod_v.at[0, pl.ds(0, CHUNK)],
                        agg_sh.at[pl.ds(s * stripe + k * CHUNK, CHUNK)])
    if rem:
        pltpu.sync_copy(prod_v.at[0, pl.ds(0, rem)],
                        agg_sh.at[pl.ds(s * stripe + nz * CHUNK, rem)])
    plsc.subcore_barrier()

    def _idx_load(g, buf, sync):
        row = tile_row0 + g * GRP
        if sync:
            pltpu.sync_copy(srcm_hbm.at[c, pl.ds(row, GRP)], idxs_v.at[buf])
            pltpu.sync_copy(dstm_hbm.at[pl.ds(row, GRP)], idxd_v.at[buf])
        else:
            pltpu.async_copy(srcm_hbm.at[c, pl.ds(row, GRP)],
                             idxs_v.at[buf], isem)
            pltpu.async_copy(dstm_hbm.at[pl.ds(row, GRP)],
                             idxd_v.at[buf], isem)

    def _fire_gather(idxrow, rbuf):
        pltpu.async_copy(h_hbm.at[idxrow], rows_v.at[rbuf], gsem)

    def _drain(sem, dst):
        pltpu.make_async_copy(h_hbm.at[pl.ds(0, CHUNK)], dst, sem).wait()

    _idx_load(0, 0, True)
    @pl.when(n_groups > 1)
    def _():
        _idx_load(1, 1, False)
    pltpu.sync_copy(w_hbm.at[pl.ds(w_row0, gw)], wv)
    _fire_gather(idxs_v.at[0, 0], 0)

    def _group(g, carry):
        pg = lax.rem(g, 2)
        for ci in range(GRP):
            rb = ci & 1
            if ci < GRP - 1:
                _fire_gather(idxs_v.at[pg, ci + 1], 1 - rb)
            else:
                @pl.when(g < n_groups - 1)
                def _():
                    pltpu.make_async_copy(srcm_hbm.at[c, pl.ds(0, GRP)],
                                          idxs_v.at[0], isem).wait()
                    pltpu.make_async_copy(dstm_hbm.at[pl.ds(0, GRP)],
                                          idxd_v.at[0], isem).wait()
                    _fire_gather(idxs_v.at[1 - pg, 0], 1 - rb)
            _drain(gsem, rows_v.at[rb])
            if ci < 2:
                @pl.when(g > 0)
                def _():
                    pltpu.make_async_copy(
                        prod_v.at[rb], agg_sh.at[pl.ds(0, CHUNK)],
                        ssem).wait()
            else:
                pltpu.make_async_copy(prod_v.at[rb],
                                      agg_sh.at[pl.ds(0, CHUNK)], ssem).wait()

            def _mul(r, _):
                prod_v[rb, r, 0:16] = (rows_v[rb, r, 0:16]
                                       * wv[r, ci * 32:ci * 32 + 16])
                prod_v[rb, r, 16:32] = (rows_v[rb, r, 16:32]
                                        * wv[r, ci * 32 + 16:ci * 32 + 32])
                return _
            lax.fori_loop(0, CHUNK, _mul, 0)
            pltpu.async_copy(prod_v.at[rb], agg_sh.at[idxd_v.at[pg, ci]],
                             ssem, add=True)
        @pl.when(g < n_groups - 2)
        def _():
            _idx_load(g + 2, pg, False)
        @pl.when(g < n_groups - 1)
        def _():
            pltpu.sync_copy(w_hbm.at[pl.ds(w_row0 + (g + 1) * gw, gw)], wv)
        return carry

    lax.fori_loop(0, n_groups, _group, 0)
    for rb in range(2):
        pltpu.make_async_copy(prod_v.at[rb], agg_sh.at[pl.ds(0, CHUNK)],
                              ssem).wait()
    plsc.subcore_barrier()

    pltpu.sync_copy(agg_sh.at[pl.ds(s * stripe, stripe)],
                    agg_hbm.at[c, pl.ds(s * stripe, stripe)])


def _edge_aggregate(ht, wpk, srcm2, dstm, n_nodes, n_edges):
    mesh = plsc.VectorSubcoreMesh(core_axis_name="c", subcore_axis_name="s",
                                  num_cores=NC, num_subcores=NS)
    body = functools.partial(_edge_sc_body, n_nodes, n_edges)
    return pl.kernel(
        body,
        out_type=jax.ShapeDtypeStruct((2, n_nodes, 32), F32),
        mesh=mesh,
        compiler_params=pltpu.CompilerParams(use_tc_tiling_on_sc=False),
        scratch_types=[
            pltpu.VMEM((2, CHUNK, 32), F32),
            pltpu.VMEM((GRP * CHUNK // 4, 128), F32),
            pltpu.VMEM((2, CHUNK, 32), F32),
            pltpu.VMEM((2, GRP, CHUNK), jnp.int32),
            pltpu.VMEM((2, GRP, CHUNK), jnp.int32),
            pltpu.SemaphoreType.DMA,
            pltpu.SemaphoreType.DMA,
            pltpu.SemaphoreType.DMA,
            pltpu.SemaphoreType.DMA,
            pltpu.VMEM_SHARED((n_nodes, 32), F32),
        ],
    )(ht, wpk.reshape(2 * (n_edges // 4), 128), srcm2, dstm)


def _prep_layer(p, c_s, c_x, d_out):
    d_in = p['l1_a'].shape[0]
    norm = 1.0 / np.sqrt(d_in * 16.0)
    wsc = jnp.concatenate([p['sc_a'], p['sc_c']], axis=1) * (c_s * norm)
    wl1 = jnp.concatenate([p['l1_a'], p['l1_c']], axis=1) * norm
    wa = jnp.concatenate([wsc, wl1], axis=2)
    wa = jnp.transpose(wa, (1, 0, 2)).reshape(32 * d_in, d_out + 64)
    wb = jnp.concatenate([p['l2_a'], p['l2_c']], axis=1) * norm
    wb = jnp.transpose(wb, (1, 0, 2)).reshape(32 * d_in, d_in)
    l3s = jnp.sum(p['l3'], axis=1) * (c_x * norm)
    wbl3 = jnp.dot(wb, l3s)
    return wa, wbl3


def kernel(x, node_attr, crystal_attr, sym_mask, edge_attr,
           edge_length_embedded, params, edge_src, edge_dst):
    n, in_dim = x.shape
    e = edge_src.shape[0]
    c_s = np.sin(np.pi / 8.0)
    c_x = np.cos(np.pi / 8.0)
    layers = params['layers']

    wa1, wbl31 = _prep_layer(layers[0], c_s, c_x, 64)
    wa2, wbl32 = _prep_layer(layers[1], c_s, c_x, 64)
    wa3, wbl33 = _prep_layer(layers[2], c_s, c_x, 1)

    w0all = jnp.concatenate([layers[0]['fc0'], layers[1]['fc0'],
                             layers[2]['fc0']], axis=1) / np.sqrt(10.0)
    w1blk = jnp.zeros((300, 192), F32)
    for l in range(3):
        w1blk = w1blk.at[100 * l:100 * (l + 1), 64 * l:64 * (l + 1)].set(
            layers[l]['fc1'] / (10.0 * 4.0))

    grid_n = n // BN
    grid_e = e // BE

    full = lambda shape: pl.BlockSpec(shape, lambda i: tuple(0 for _ in shape))

    ycat, s1, h1 = pl.pallas_call(
        _embed_a1_body,
        grid=(grid_n,),
        in_specs=[
            pl.BlockSpec((BN, in_dim), lambda i: (i, 0)),
            pl.BlockSpec((BN, 16), lambda i: (i, 0)),
            pl.BlockSpec((BN, 16), lambda i: (i, 0)),
            full((in_dim, 64)), full((16, 16)), full((16, 16)),
            full((2048, 128)),
        ],
        out_specs=[
            pl.BlockSpec((BN, 32), lambda i: (i, 0)),
            pl.BlockSpec((BN, 64), lambda i: (i, 0)),
            pl.BlockSpec((2, BN, 32), lambda i: (0, i, 0)),
        ],
        out_shape=[
            jax.ShapeDtypeStruct((n, 32), F32),
            jax.ShapeDtypeStruct((n, 64), F32),
            jax.ShapeDtypeStruct((2, n, 32), F32),
        ],
    )(x, node_attr, crystal_attr, params['em_w'], params['ema_w'],
      params['emc_w'], wa1)

    w1, w2, w3 = pl.pallas_call(
        _radial_body,
        grid=(grid_e,),
        in_specs=[
            pl.BlockSpec((BE, 10), lambda i: (i, 0)),
            full((10, 300)), full((300, 192)),
        ],
        out_specs=[pl.BlockSpec((2, BE // 4, 128), lambda i: (0, i, 0))] * 3,
        out_shape=[jax.ShapeDtypeStruct((2, e // 4, 128), F32)] * 3,
    )(edge_length_embedded, w0all, w1blk)

    src32 = edge_src.astype(jnp.int32)
    srcm2 = jnp.stack([src32, src32 + n]).reshape(2, e // CHUNK, CHUNK)
    dstm = jnp.reshape(edge_dst.astype(jnp.int32), (e // CHUNK, CHUNK))

    def conv_node(agg, s_prev, wbl3, wanext, s_width):
        return pl.pallas_call(
            functools.partial(_conv_node_body, s_width=s_width),
            grid=(grid_n,),
            in_specs=[
                pl.BlockSpec((2, BN, 32), lambda i: (0, i, 0)),
                pl.BlockSpec((BN, 64), lambda i: (i, 0)),
                pl.BlockSpec((BN, 32), lambda i: (i, 0)),
                full((2048, 64)), full((2048, s_width + 64)),
            ],
            out_specs=[
                pl.BlockSpec((BN, s_width), lambda i: (i, 0)),
                pl.BlockSpec((2, BN, 32), lambda i: (0, i, 0)),
            ],
            out_shape=[
                jax.ShapeDtypeStruct((n, s_width), F32),
                jax.ShapeDtypeStruct((2, n, 32), F32),
            ],
        )(agg, s_prev, ycat, wbl3, wanext)

    agg = _edge_aggregate(h1.reshape(2 * n, 32), w1,
                          srcm2, dstm, n, e)
    s2, h2 = conv_node(agg, s1, wbl31, wa2, 64)

    agg = _edge_aggregate(h2.reshape(2 * n, 32), w2,
                          srcm2, dstm, n, e)
    s3, h3 = conv_node(agg, s2, wbl32, wa3, 1)

    agg = _edge_aggregate(h3.reshape(2 * n, 32), w3,
                          srcm2, dstm, n, e)
    out = pl.pallas_call(
        functools.partial(_final_body, inv_sqrt_n=1.0 / np.sqrt(float(n))),
        grid=(grid_n,),
        in_specs=[
            pl.BlockSpec((2, BN, 32), lambda i: (0, i, 0)),
            pl.BlockSpec((BN, 1), lambda i: (i, 0)),
            pl.BlockSpec((BN, 32), lambda i: (i, 0)),
            full((2048, 1)),
        ],
        out_specs=pl.BlockSpec((1, 1), lambda i: (0, 0)),
        out_shape=jax.ShapeDtypeStruct((1, 1), F32),
    )(agg, s3, ycat, wbl33)
    return out


# speedup vs baseline: 2.0322x; 1.2698x over previous
import functools

import numpy as np
import jax
import jax.numpy as jnp
from jax import lax
from jax.experimental import pallas as pl
from jax.experimental.pallas import tpu as pltpu
from jax.experimental.pallas import tpu_sc as plsc

F32 = jnp.float32

BN = 400
BE = 4000
CHUNK = 100
GRP = 4
NC, NS = 2, 16

C_S = np.sin(np.pi / 8.0)
C_X = np.cos(np.pi / 8.0)
R10 = np.sqrt(10.0)
INV32 = np.float32(1.0 / 32.0)


def _silu(v):
    return v * jax.nn.sigmoid(v)


def _red(y, t, dk, nj=16):
    acc = y[:, 0:1] * t[:, 0:dk]
    for j in range(1, nj):
        acc = acc + y[:, j:j + 1] * t[:, j * dk:(j + 1) * dk]
    return acc


def _sumslices(t, dk, nj=16):
    acc = t[:, 0:dk]
    for j in range(1, nj):
        acc = acc + t[:, j * dk:(j + 1) * dk]
    return acc


def _dot(a, b):
    return lax.dot_general(a, b, (((1,), (0,)), ((), ())),
                           preferred_element_type=F32)


def _embed_a1_body(x_ref, nat_ref, cat_ref, emw_ref, emaw_ref, emcw_ref,
                   wt_ref, ycat_ref, s1_ref, h1_ref):
    z = _dot(x_ref[...], emw_ref[...])
    na = _dot(nat_ref[...], emaw_ref[...])
    ca = _dot(cat_ref[...], emcw_ref[...])
    y = jnp.concatenate([na, ca], axis=1)
    ycat_ref[...] = y
    t = _dot(z, wt_ref[...])
    s_a = _red(na, t[:, 0:1024], 64) * INV32
    s_c = _red(ca, t[:, 1024:2048], 64) * INV32
    h_a = _red(na, t[:, 2048:3072], 64) * INV32
    h_c = _red(ca, t[:, 3072:4096], 64) * INV32
    s1_ref[...] = s_a + s_c
    h = h_a + h_c
    h1_ref[0] = h[:, 0:32]
    h1_ref[1] = h[:, 32:64]


def _radial_body(ele_ref, w0_ref, w1blk_ref, w1_ref, w2_ref, w3_ref):
    h = _silu(_dot(ele_ref[...], w0_ref[...]) / R10)
    wall = (_dot(h, w1blk_ref[...]) / 10.0) * 0.25
    nq = wall.shape[0] // (4 * CHUNK)
    for l, wref in enumerate((w1_ref, w2_ref, w3_ref)):
        for c in range(2):
            col = 64 * l + 32 * c
            pieces = [
                jnp.concatenate(
                    [wall[q * 4 * CHUNK + t * CHUNK:
                          q * 4 * CHUNK + (t + 1) * CHUNK, col:col + 32]
                     for q in range(nq)], axis=0)
                for t in range(4)
            ]
            wref[c] = jnp.concatenate(pieces, axis=1)


def _conv_node_body(agg_ref, s_ref, ycat_ref, wb_ref, l3_ref, wtn_ref,
                    snext_ref, hnext_ref, *, s_width):
    y = ycat_ref[...]
    na = y[:, 0:16]
    ca = y[:, 16:32]
    agg = jnp.concatenate([agg_ref[0], agg_ref[1]], axis=1)
    t2 = _dot(agg, wb_ref[...])
    h2i = (_red(na, t2[:, 0:1024], 64) * INV32
           + _red(ca, t2[:, 1024:2048], 64) * INV32)
    t3 = _dot(h2i, l3_ref[...])
    h2 = _sumslices(t3, 64) * INV32
    zn = _silu(C_S * s_ref[...] + C_X * h2)
    tn = _dot(zn, wtn_ref[...])
    ds = 16 * s_width
    s_a = _red(na, tn[:, 0:ds], s_width) * INV32
    s_c = _red(ca, tn[:, ds:2 * ds], s_width) * INV32
    h_a = _red(na, tn[:, 2 * ds:2 * ds + 1024], 64) * INV32
    h_c = _red(ca, tn[:, 2 * ds + 1024:2 * ds + 2048], 64) * INV32
    snext_ref[...] = s_a + s_c
    h = h_a + h_c
    hnext_ref[0] = h[:, 0:32]
    hnext_ref[1] = h[:, 32:64]


def _final_body(agg_ref, s3_ref, ycat_ref, wb_ref, l3_ref, out_ref, *,
                inv_sqrt_n):
    i = pl.program_id(0)
    y = ycat_ref[...]
    na = y[:, 0:16]
    ca = y[:, 16:32]
    agg = jnp.concatenate([agg_ref[0], agg_ref[1]], axis=1)
    t2 = _dot(agg, wb_ref[...])
    h2i = (_red(na, t2[:, 0:1024], 64) * INV32
           + _red(ca, t2[:, 1024:2048], 64) * INV32)
    t3 = _dot(h2i, l3_ref[...])
    h2 = _sumslices(t3, 1) * INV32
    z3 = C_S * s3_ref[...] + C_X * h2
    part = jnp.sum(z3) * inv_sqrt_n

    @pl.when(i == 0)
    def _():
        out_ref[...] = jnp.zeros_like(out_ref)

    out_ref[...] += part[None, None]


def _edge_sc_body(n_nodes, n_edges,
                  h_hbm, w_hbm, srcm_hbm, dstm_hbm, agg_hbm,
                  rows_v, wv, idxs_v, idxd_v,
                  gsem, wsem, isem, ssem, agg_sh):
    c = lax.axis_index("c")
    s = lax.axis_index("s")
    rows_per_tile = n_edges // CHUNK // NS
    n_groups = rows_per_tile // GRP
    tile_row0 = s * rows_per_tile
    w_row0 = c * (n_edges // 4) + tile_row0 * (CHUNK // 4)
    gw = GRP * (CHUNK // 4)
    stripe = n_nodes // NS

    def _zero_body(r, zc):
        rows_v[0, r, 0:16] = jnp.zeros((16,), F32)
        rows_v[0, r, 16:32] = jnp.zeros((16,), F32)
        return zc
    lax.fori_loop(0, CHUNK, _zero_body, 0)
    nz = stripe // CHUNK
    rem = stripe - nz * CHUNK
    for k in range(nz):
        pltpu.sync_copy(rows_v.at[0, pl.ds(0, CHUNK)],
                        agg_sh.at[pl.ds(s * stripe + k * CHUNK, CHUNK)])
    if rem:
        pltpu.sync_copy(rows_v.at[0, pl.ds(0, rem)],
                        agg_sh.at[pl.ds(s * stripe + nz * CHUNK, rem)])
    plsc.subcore_barrier()

    def _idx_load(g, buf, sync):
        row = tile_row0 + g * GRP
        if sync:
            pltpu.sync_copy(srcm_hbm.at[c, pl.ds(row, GRP)], idxs_v.at[buf])
            pltpu.sync_copy(dstm_hbm.at[pl.ds(row, GRP)], idxd_v.at[buf])
        else:
            pltpu.async_copy(srcm_hbm.at[c, pl.ds(row, GRP)],
                             idxs_v.at[buf], isem)
            pltpu.async_copy(dstm_hbm.at[pl.ds(row, GRP)],
                             idxd_v.at[buf], isem)

    def _fire_gather(idxrow, rbuf):
        pltpu.async_copy(h_hbm.at[idxrow], rows_v.at[rbuf], gsem)

    def _drain(sem, dst):
        pltpu.make_async_copy(h_hbm.at[pl.ds(0, CHUNK)], dst, sem).wait()

    _idx_load(0, 0, True)
    @pl.when(n_groups > 1)
    def _():
        _idx_load(1, 1, False)
    pltpu.sync_copy(w_hbm.at[pl.ds(w_row0, gw)], wv)
    _fire_gather(idxs_v.at[0, 0], 0)
    _fire_gather(idxs_v.at[0, 1], 1)

    def _group(g, carry):
        pg = lax.rem(g, 2)
        for ci in range(GRP):
            if ci < 2:
                @pl.when(g > 0)
                def _():
                    pltpu.make_async_copy(rows_v.at[0],
                                          agg_sh.at[pl.ds(0, CHUNK)],
                                          ssem).wait()
            else:
                pltpu.make_async_copy(rows_v.at[0],
                                      agg_sh.at[pl.ds(0, CHUNK)], ssem).wait()
            if ci < 2:
                _fire_gather(idxs_v.at[pg, ci + 2], (ci + 2) % 4)
            else:
                @pl.when(g < n_groups - 1)
                def _():
                    if ci == 2:
                        pltpu.make_async_copy(srcm_hbm.at[c, pl.ds(0, GRP)],
                                              idxs_v.at[0], isem).wait()
                        pltpu.make_async_copy(dstm_hbm.at[pl.ds(0, GRP)],
                                              idxd_v.at[0], isem).wait()
                    _fire_gather(idxs_v.at[1 - pg, ci - 2], (ci + 2) % 4)
            _drain(gsem, rows_v.at[ci])

            def _mul(r, carry2):
                rows_v[ci, r, 0:16] = (rows_v[ci, r, 0:16]
                                       * wv[r, ci * 32:ci * 32 + 16])
                rows_v[ci, r, 16:32] = (rows_v[ci, r, 16:32]
                                        * wv[r, ci * 32 + 16:ci * 32 + 32])
                return carry2
            lax.fori_loop(0, CHUNK, _mul, 0)
            pltpu.async_copy(rows_v.at[ci], agg_sh.at[idxd_v.at[pg, ci]],
                             ssem, add=True)
        @pl.when(g < n_groups - 2)
        def _():
            _idx_load(g + 2, pg, False)
        @pl.when(g < n_groups - 1)
        def _():
            pltpu.sync_copy(w_hbm.at[pl.ds(w_row0 + (g + 1) * gw, gw)], wv)
        return carry

    lax.fori_loop(0, n_groups, _group, 0)
    for _k in range(2):
        pltpu.make_async_copy(rows_v.at[0], agg_sh.at[pl.ds(0, CHUNK)],
                              ssem).wait()
    plsc.subcore_barrier()

    pltpu.sync_copy(agg_sh.at[pl.ds(s * stripe, stripe)],
                    agg_hbm.at[c, pl.ds(s * stripe, stripe)])


def _edge_aggregate(ht, wpk, srcm2, dstm, n_nodes, n_edges):
    mesh = plsc.VectorSubcoreMesh(core_axis_name="c", subcore_axis_name="s",
                                  num_cores=NC, num_subcores=NS)
    body = functools.partial(_edge_sc_body, n_nodes, n_edges)
    return pl.kernel(
        body,
        out_type=jax.ShapeDtypeStruct((2, n_nodes, 32), F32),
        mesh=mesh,
        compiler_params=pltpu.CompilerParams(use_tc_tiling_on_sc=False),
        scratch_types=[
            pltpu.VMEM((4, CHUNK, 32), F32),
            pltpu.VMEM((GRP * CHUNK // 4, 128), F32),
            pltpu.VMEM((2, GRP, CHUNK), jnp.int32),
            pltpu.VMEM((2, GRP, CHUNK), jnp.int32),
            pltpu.SemaphoreType.DMA,
            pltpu.SemaphoreType.DMA,
            pltpu.SemaphoreType.DMA,
            pltpu.SemaphoreType.DMA,
            pltpu.VMEM_SHARED((n_nodes, 32), F32),
        ],
    )(ht, wpk.reshape(2 * (n_edges // 4), 128), srcm2, dstm)


def _flat(w):
    return w.reshape(w.shape[0], -1)


def kernel(x, node_attr, crystal_attr, sym_mask, edge_attr,
           edge_length_embedded, params, edge_src, edge_dst):
    n, in_dim = x.shape
    e = edge_src.shape[0]
    layers = params['layers']

    def wt_a(p):
        return jnp.concatenate([_flat(p['sc_a']), _flat(p['sc_c']),
                                _flat(p['l1_a']), _flat(p['l1_c'])], axis=1)

    def wt_b(p):
        return jnp.concatenate([_flat(p['l2_a']), _flat(p['l2_c'])], axis=1)

    wt1 = wt_a(layers[0])
    wt2 = wt_a(layers[1])
    wt3 = wt_a(layers[2])
    wb1, l31 = wt_b(layers[0]), _flat(layers[0]['l3'])
    wb2, l32 = wt_b(layers[1]), _flat(layers[1]['l3'])
    wb3, l33 = wt_b(layers[2]), _flat(layers[2]['l3'])

    w0all = jnp.concatenate([layers[0]['fc0'], layers[1]['fc0'],
                             layers[2]['fc0']], axis=1)
    w1blk = jnp.zeros((300, 192), F32)
    for l in range(3):
        w1blk = w1blk.at[100 * l:100 * (l + 1), 64 * l:64 * (l + 1)].set(
            layers[l]['fc1'])

    grid_n = n // BN
    grid_e = e // BE

    full = lambda shape: pl.BlockSpec(shape, lambda i: tuple(0 for _ in shape))

    ycat, s1, h1 = pl.pallas_call(
        _embed_a1_body,
        grid=(grid_n,),
        in_specs=[
            pl.BlockSpec((BN, in_dim), lambda i: (i, 0)),
            pl.BlockSpec((BN, 16), lambda i: (i, 0)),
            pl.BlockSpec((BN, 16), lambda i: (i, 0)),
            full((in_dim, 64)), full((16, 16)), full((16, 16)),
            full((64, 4096)),
        ],
        out_specs=[
            pl.BlockSpec((BN, 32), lambda i: (i, 0)),
            pl.BlockSpec((BN, 64), lambda i: (i, 0)),
            pl.BlockSpec((2, BN, 32), lambda i: (0, i, 0)),
        ],
        out_shape=[
            jax.ShapeDtypeStruct((n, 32), F32),
            jax.ShapeDtypeStruct((n, 64), F32),
            jax.ShapeDtypeStruct((2, n, 32), F32),
        ],
    )(x, node_attr, crystal_attr, params['em_w'], params['ema_w'],
      params['emc_w'], wt1)

    w1, w2, w3 = pl.pallas_call(
        _radial_body,
        grid=(grid_e,),
        in_specs=[
            pl.BlockSpec((BE, 10), lambda i: (i, 0)),
            full((10, 300)), full((300, 192)),
        ],
        out_specs=[pl.BlockSpec((2, BE // 4, 128), lambda i: (0, i, 0))] * 3,
        out_shape=[jax.ShapeDtypeStruct((2, e // 4, 128), F32)] * 3,
    )(edge_length_embedded, w0all, w1blk)

    src32 = edge_src.astype(jnp.int32)
    srcm2 = jnp.stack([src32, src32 + n]).reshape(2, e // CHUNK, CHUNK)
    dstm = jnp.reshape(edge_dst.astype(jnp.int32), (e // CHUNK, CHUNK))

    def conv_node(agg, s_prev, wb, l3f, wtn, s_width):
        return pl.pallas_call(
            functools.partial(_conv_node_body, s_width=s_width),
            grid=(grid_n,),
            in_specs=[
                pl.BlockSpec((2, BN, 32), lambda i: (0, i, 0)),
                pl.BlockSpec((BN, 64), lambda i: (i, 0)),
                pl.BlockSpec((BN, 32), lambda i: (i, 0)),
                full((64, 2048)), full((64, 1024)),
                full((64, 2 * 16 * s_width + 2048)),
            ],
            out_specs=[
                pl.BlockSpec((BN, s_width), lambda i: (i, 0)),
                pl.BlockSpec((2, BN, 32), lambda i: (0, i, 0)),
            ],
            out_shape=[
                jax.ShapeDtypeStruct((n, s_width), F32),
                jax.ShapeDtypeStruct((2, n, 32), F32),
            ],
        )(agg, s_prev, ycat, wb, l3f, wtn)

    agg = _edge_aggregate(h1.reshape(2 * n, 32), w1, srcm2, dstm, n, e)
    s2, h2 = conv_node(agg, s1, wb1, l31, wt2, 64)

    agg = _edge_aggregate(h2.reshape(2 * n, 32), w2, srcm2, dstm, n, e)
    s3, h3 = conv_node(agg, s2, wb2, l32, wt3, 1)

    agg = _edge_aggregate(h3.reshape(2 * n, 32), w3, srcm2, dstm, n, e)
    out = pl.pallas_call(
        functools.partial(_final_body, inv_sqrt_n=1.0 / np.sqrt(float(n))),
        grid=(grid_n,),
        in_specs=[
            pl.BlockSpec((2, BN, 32), lambda i: (0, i, 0)),
            pl.BlockSpec((BN, 1), lambda i: (i, 0)),
            pl.BlockSpec((BN, 32), lambda i: (i, 0)),
            full((64, 2048)), full((64, 16)),
        ],
        out_specs=pl.BlockSpec((1, 1), lambda i: (0, 0)),
        out_shape=jax.ShapeDtypeStruct((1, 1), F32),
    )(agg, s3, ycat, wb3, l33)
    return out
